# baseline (speedup 1.0000x reference)
import functools

import jax
import jax.numpy as jnp
from jax import lax
from jax.experimental import pallas as pl
from jax.experimental.pallas import tpu as pltpu
from jax.experimental.pallas import tpu_sc as plsc

_NC = 2
_NS = 16
_NW = _NC * _NS
_L = 16

_BB = 128


@functools.lru_cache(maxsize=None)
def _build(batch, fields, dim):
    nj8 = fields * dim // 8
    nblocks = batch // _BB
    bpw = nblocks // _NW
    assert bpw % 2 == 0
    rpb = _BB * fields
    nstr = rpb // _BB
    mesh = plsc.VectorSubcoreMesh(core_axis_name="c", subcore_axis_name="s")

    @functools.partial(
        pl.kernel,
        out_type=jax.ShapeDtypeStruct((nj8, nblocks, 8, _BB), jnp.float32),
        mesh=mesh,
        scratch_types=[
            pltpu.VMEM((2, rpb), jnp.int32),
            pltpu.VMEM((rpb,), jnp.int32),
            pltpu.VMEM((rpb, dim), jnp.float32),
            pltpu.VMEM((2, 1, 1, 8, _BB), jnp.float32),
            pltpu.SemaphoreType.DMA,
            pltpu.SemaphoreType.DMA,
            pltpu.SemaphoreType.DMA,
            pltpu.SemaphoreType.DMA,
            pltpu.SemaphoreType.DMA,
        ],
        compiler_params=pltpu.CompilerParams(
            use_tc_tiling_on_sc=False, needs_layout_passes=False),
    )
    def gather_kernel(idx_hbm, off_hbm, table_hbm, out_hbm,
                      idx_v, off_v, rows_v, tile_v,
                      sem_in0, sem_in1, sem_g, sem_wb0, sem_wb1):
        sem_in = (sem_in0, sem_in1)
        sem_wb = (sem_wb0, sem_wb1)
        wid = lax.axis_index("s") * _NC + lax.axis_index("c")
        bb0 = wid * bpw
        pltpu.sync_copy(off_hbm, off_v)
        lanes = lax.broadcasted_iota(jnp.int32, (_L,), 0)
        lanes_f = lanes * fields

        def issue_in(bb, p):
            pltpu.async_copy(idx_hbm.at[bb], idx_v.at[p], sem_in[p])

        def wait_in(p):
            pltpu.make_async_copy(idx_hbm.at[0], idx_v.at[p],
                                  sem_in[p]).wait()

        def adds(p):
            def grp(i, carry):
                s = pl.ds(i * _L, _L)
                idx_v[p, s] = idx_v[p, s] + off_v[s]
                return carry
            lax.fori_loop(0, rpb // _L, grp, 0)

        def gathers(p):
            for q in range(nstr):
                pltpu.async_copy(
                    table_hbm.at[idx_v.at[p, pl.ds(q * _BB, _BB)]],
                    rows_v.at[pl.ds(q * _BB, _BB)], sem_g)
            for q in range(nstr):
                pltpu.make_async_copy(
                    table_hbm.at[idx_v.at[p, pl.ds(q * _BB, _BB)]],
                    rows_v.at[pl.ds(q * _BB, _BB)], sem_g).wait()

        def wait_tile(h):
            pltpu.make_async_copy(
                tile_v.at[h], out_hbm.at[pl.ds(0, 1), pl.ds(0, 1)],
                sem_wb[h]).wait()

        def transpose_block(bb):
            def utile(u, carry):
                for h in (0, 1):
                    j8 = 2 * u + h
                    f = j8 >> 2
                    d0 = (j8 & 3) * 8

                    @pl.when(u >= 1)
                    def _():
                        wait_tile(h)
                    for jj in range(8):
                        col = jnp.zeros((_L,), jnp.int32) + (d0 + jj)
                        for k in range(_BB // _L):
                            row = lanes_f + (k * _L * fields + f)
                            tile_v[h, 0, 0, jj, pl.ds(k * _L, _L)] = (
                                plsc.load_gather(rows_v, [row, col]))
                    pltpu.async_copy(
                        tile_v.at[h],
                        out_hbm.at[pl.ds(j8, 1), pl.ds(bb, 1)], sem_wb[h])
                return carry
            lax.fori_loop(0, nj8 // 2, utile, 0)
            wait_tile(0)
            wait_tile(1)

        issue_in(bb0, 0)
        issue_in(bb0 + 1, 1)

        def body(t, carry):
            for p in (0, 1):
                bb = bb0 + 2 * t + p
                wait_in(p)
                adds(p)
                gathers(p)
                @pl.when(t < (bpw // 2) - 1)
                def _():
                    issue_in(bb + 2, p)
                transpose_block(bb)
            return carry

        lax.fori_loop(0, bpw // 2, body, 0)

    return gather_kernel


def kernel(input, num_classes, table):
    batch, fields = input.shape
    dim = table.shape[1]
    offsets = jnp.concatenate([
        jnp.zeros((1,), dtype=num_classes.dtype),
        jnp.cumsum(num_classes)[:-1],
    ]).astype(jnp.int32)
    idx2 = input.reshape(batch // _BB, _BB * fields)
    offc = jnp.broadcast_to(offsets, (_BB, fields)).reshape(_BB * fields)
    out4 = _build(batch, fields, dim)(idx2, offc, table)
    return out4.transpose(1, 3, 0, 2).reshape(batch, fields * dim)
